# baseline (speedup 1.0000x reference)
import functools

import numpy as np

import jax
import jax.numpy as jnp
from jax import lax
from jax.experimental import pallas as pl
from jax.experimental.pallas import tpu as pltpu
from jax.experimental.pallas import tpu_sc as plsc

N = 10000
E = 160000
D_IN = 256
D_OUT = 256
L = 16
NVEC = D_OUT // L
NPAIR = D_OUT // 32
NC, NS = 2, 16
NW = NC * NS
NPW = 320
WSZ = 344
WLAST = ((N + 1 - WSZ) // 8) * 8
PBUF = 360
CHUNK = 256
QH = 80

_MM_BLK = 1000


def _matmul_body(x_ref, w_ref, o_ref):
    y = jnp.dot(x_ref[...], w_ref[...], preferred_element_type=jnp.float32)
    lo = lax.bitcast_convert_type(
        y[:, :128].astype(jnp.bfloat16), jnp.uint16).astype(jnp.uint32)
    hi = lax.bitcast_convert_type(
        y[:, 128:].astype(jnp.bfloat16), jnp.uint16).astype(jnp.uint32)
    o_ref[...] = lo | (hi << 16)


def _matmul(x, w):
    return pl.pallas_call(
        _matmul_body,
        grid=(N // _MM_BLK,),
        in_specs=[
            pl.BlockSpec((_MM_BLK, D_IN), lambda i: (i, 0)),
            pl.BlockSpec((D_IN, D_OUT), lambda i: (0, 0)),
        ],
        out_specs=pl.BlockSpec((_MM_BLK, D_OUT // 2), lambda i: (i, 0)),
        out_shape=jax.ShapeDtypeStruct((N, D_OUT // 2), jnp.uint32),
    )(x, w)


def _sc_body(xw_hbm, ptr_hbm, edge_hbm, out_hbm, ptr_vm, idx_v, gbuf, qbuf,
             sem_g, sem_i, sem_o):
    cid = lax.axis_index("c")
    sid = lax.axis_index("s")
    w = sid * NC + cid
    n0 = pl.multiple_of(jnp.minimum(w * NPW, N - NPW), 8)
    wstart = pl.multiple_of(jnp.minimum(n0, WLAST), 8)
    off = n0 - wstart
    pltpu.sync_copy(ptr_hbm.at[pl.ds(wstart, WSZ)], ptr_vm.at[pl.ds(0, WSZ)])

    @pl.when(off > 0)
    def _():
        ptr_vm[pl.ds(off + NPW, L)] = jnp.full((L,), E, jnp.int32)

    def ptr_at(j):
        return ptr_vm[pl.ds(off + j, L)][0]

    lo = ptr_at(0)
    hi = ptr_at(NPW)
    lo8 = lo - lax.rem(lo, 8)
    nchunks = lax.div(hi - lo8 + (CHUNK - 1), CHUNK)
    nt = 2 * jnp.maximum(lax.div(nchunks + 1, 2), 1)
    zero_acc = tuple(jnp.zeros((L,), jnp.float32) for _ in range(NVEC))

    def dma_base(t):
        return pl.multiple_of(
            jnp.minimum(lo8 + t * CHUNK, E - CHUNK), 8)

    def idx_copy(t, b):
        return pltpu.make_async_copy(
            edge_hbm.at[pl.ds(dma_base(t), CHUNK)],
            idx_v.at[pl.ds(b * CHUNK, CHUNK)], sem_i.at[b])

    def gather(b):
        return pltpu.make_async_copy(
            xw_hbm.at[idx_v.at[pl.ds(b * CHUNK, CHUNK)]], gbuf.at[b],
            sem_g.at[b])

    def gather_drain(b):
        return pltpu.make_async_copy(
            xw_hbm.at[pl.ds(0, CHUNK)], gbuf.at[b], sem_g.at[b])

    idx_copy(0, 0).start()
    idx_copy(1, 1).start()
    idx_copy(0, 0).wait()
    gather(0).start()

    def accum(acc, par, bdma, p0, p1):
        def body(e, acc):
            r = e - bdma
            out = list(acc)
            for d in range(NPAIR):
                wv = gbuf[par, r, pl.ds(d * L, L)]
                lo = plsc.bitcast(wv << 16, jnp.float32)
                hi = plsc.bitcast(wv, jnp.float32)
                out[d] = out[d] + lo
                out[d + NPAIR] = out[d + NPAIR] + hi
            return tuple(out)
        return plsc.parallel_loop(p0, p1, 1, unroll=3, carry=acc)(body)

    def qflush(q, par2):
        return pltpu.make_async_copy(
            qbuf.at[par2], out_hbm.at[pl.ds(n0 + q * QH, QH)], sem_o.at[par2])

    def store_row(i, acc):
        q = lax.div(i, QH)
        par2 = lax.rem(q, 2)
        io = i - q * QH
        for d in range(NVEC):
            qbuf[par2, io, pl.ds(d * L, L)] = acc[d]

        @pl.when(io == QH - 1)
        def _():
            qflush(q, par2).start()

            @pl.when(q > 0)
            def _():
                qflush(0, 1 - par2).wait()

    def chunk_body(t, carry):
        par = lax.rem(t, 2)
        nxt = 1 - par
        idx_copy(t + 1, nxt).wait()
        gather(nxt).start()
        gather_drain(par).wait()
        idx_copy(t + 2, par).start()

        base = lo8 + t * CHUNK
        end = base + CHUNK
        bdma = dma_base(t)

        def wcond(st):
            return jnp.logical_and(st[0] < NPW, ptr_at(st[0] + 1) <= end)

        def wbody(st):
            i = st[0]
            p0 = jnp.maximum(ptr_at(i), base)
            p1 = ptr_at(i + 1)
            acc = accum(st[1:], par, bdma, p0, p1)
            store_row(i, acc)
            return (i + 1,) + zero_acc

        st = lax.while_loop(wcond, wbody, carry)
        i = st[0]
        ic = jnp.minimum(i, NPW - 1)
        p0 = jnp.maximum(ptr_at(ic), base)
        p1 = jnp.where(i < NPW, jnp.minimum(ptr_at(ic + 1), end), p0)
        p0 = jnp.minimum(p0, p1)
        acc = accum(st[1:], par, bdma, p0, p1)
        return (i,) + acc

    st = lax.fori_loop(0, nt, chunk_body, (jnp.int32(0),) + zero_acc)

    gather_drain(0).wait()
    idx_copy(0, 1).wait()

    def fcond(st):
        return st[0] < NPW

    def fbody(st):
        store_row(st[0], st[1:])
        return (st[0] + 1,) + zero_acc

    lax.while_loop(fcond, fbody, st)
    qflush(0, 1).wait()


@jax.jit
def _sc_gnn(xw, ptr_pad, edge_pad):
    kern = pl.kernel(
        _sc_body,
        out_type=jax.ShapeDtypeStruct((N, D_OUT), jnp.float32),
        mesh=plsc.VectorSubcoreMesh(core_axis_name="c", subcore_axis_name="s"),
        compiler_params=pltpu.CompilerParams(needs_layout_passes=False),
        scratch_types=[
            pltpu.VMEM((PBUF,), jnp.int32),
            pltpu.VMEM((2 * CHUNK,), jnp.int32),
            pltpu.VMEM((2, CHUNK, D_OUT // 2), jnp.uint32),
            pltpu.VMEM((2, QH, D_OUT), jnp.float32),
            pltpu.SemaphoreType.DMA((2,)),
            pltpu.SemaphoreType.DMA((2,)),
            pltpu.SemaphoreType.DMA((2,)),
        ],
    )
    return kern(xw, ptr_pad, edge_pad)


def kernel(X, nodePointer, edgeList, blockPartition, edgeToColumn, edgeToRow,
           adj_coo, block_num, g_nodes, weights):
    XW = _matmul(X, weights)
    return _sc_gnn(XW, nodePointer, edgeList)


# speedup vs baseline: 1.0146x; 1.0146x over previous
import functools

import numpy as np

import jax
import jax.numpy as jnp
from jax import lax
from jax.experimental import pallas as pl
from jax.experimental.pallas import tpu as pltpu
from jax.experimental.pallas import tpu_sc as plsc

N = 10000
E = 160000
D_IN = 256
D_OUT = 256
L = 16
NVEC = D_OUT // L
NPAIR = D_OUT // 32
NC, NS = 2, 16
NW = NC * NS
NPW = 320
WSZ = 344
WLAST = ((N + 1 - WSZ) // 8) * 8
PBUF = 360
CHUNK = 256
QH = 80

_MM_BLK = 1000


def _matmul_body(x_ref, w_ref, o_ref):
    y = jnp.dot(x_ref[...], w_ref[...], preferred_element_type=jnp.float32)
    lo = lax.bitcast_convert_type(
        y[:, :128].astype(jnp.bfloat16), jnp.uint16).astype(jnp.uint32)
    hi = lax.bitcast_convert_type(
        y[:, 128:].astype(jnp.bfloat16), jnp.uint16).astype(jnp.uint32)
    o_ref[...] = lo | (hi << 16)


def _matmul(x, w):
    return pl.pallas_call(
        _matmul_body,
        grid=(N // _MM_BLK,),
        in_specs=[
            pl.BlockSpec((_MM_BLK, D_IN), lambda i: (i, 0)),
            pl.BlockSpec((D_IN, D_OUT), lambda i: (0, 0)),
        ],
        out_specs=pl.BlockSpec((_MM_BLK, D_OUT // 2), lambda i: (i, 0)),
        out_shape=jax.ShapeDtypeStruct((N, D_OUT // 2), jnp.uint32),
    )(x, w)


def _sc_body(xw_hbm, ptr_hbm, edge_hbm, out_hbm, ptr_vm, idx_v, gbuf, qbuf,
             sem_g, sem_i, sem_o):
    cid = lax.axis_index("c")
    sid = lax.axis_index("s")
    w = sid * NC + cid
    n0 = pl.multiple_of(jnp.minimum(w * NPW, N - NPW), 8)
    wstart = pl.multiple_of(jnp.minimum(n0, WLAST), 8)
    off = n0 - wstart
    pltpu.sync_copy(ptr_hbm.at[pl.ds(wstart, WSZ)], ptr_vm.at[pl.ds(0, WSZ)])

    @pl.when(off > 0)
    def _():
        ptr_vm[pl.ds(off + NPW, L)] = jnp.full((L,), E, jnp.int32)

    def ptr_at(j):
        return ptr_vm[pl.ds(off + j, L)][0]

    lo = ptr_at(0)
    hi = ptr_at(NPW)
    lo8 = lo - lax.rem(lo, 8)
    nchunks = lax.div(hi - lo8 + (CHUNK - 1), CHUNK)
    nt = 2 * jnp.maximum(lax.div(nchunks + 1, 2), 1)
    zero_acc = tuple(jnp.zeros((L,), jnp.float32) for _ in range(NVEC))

    def dma_base(t):
        return pl.multiple_of(
            jnp.minimum(lo8 + t * CHUNK, E - CHUNK), 8)

    def idx_copy(t, b):
        return pltpu.make_async_copy(
            edge_hbm.at[pl.ds(dma_base(t), CHUNK)],
            idx_v.at[pl.ds(b * CHUNK, CHUNK)], sem_i.at[b])

    def gather(b):
        return pltpu.make_async_copy(
            xw_hbm.at[idx_v.at[pl.ds(b * CHUNK, CHUNK)]], gbuf.at[b],
            sem_g.at[b])

    def gather_drain(b):
        return pltpu.make_async_copy(
            xw_hbm.at[pl.ds(0, CHUNK)], gbuf.at[b], sem_g.at[b])

    idx_copy(0, 0).start()
    idx_copy(1, 1).start()
    idx_copy(0, 0).wait()
    gather(0).start()

    def accum(acc, par, bdma, p0, p1):
        def body(e, acc):
            r = e - bdma
            out = list(acc)
            for d in range(NPAIR):
                wv = gbuf[par, r, pl.ds(d * L, L)]
                lo = plsc.bitcast(wv << 16, jnp.float32)
                hi = plsc.bitcast(wv, jnp.float32)
                out[d] = out[d] + lo
                out[d + NPAIR] = out[d + NPAIR] + hi
            return tuple(out)
        return plsc.parallel_loop(p0, p1, 1, unroll=2, carry=acc)(body)

    def qflush(q, par2):
        return pltpu.make_async_copy(
            qbuf.at[par2], out_hbm.at[pl.ds(n0 + q * QH, QH)], sem_o.at[par2])

    def store_row(i, acc):
        q = lax.div(i, QH)
        par2 = lax.rem(q, 2)
        io = i - q * QH
        for d in range(NVEC):
            qbuf[par2, io, pl.ds(d * L, L)] = acc[d]

        @pl.when(io == QH - 1)
        def _():
            qflush(q, par2).start()

            @pl.when(q > 0)
            def _():
                qflush(0, 1 - par2).wait()

    def chunk_body(t, carry):
        par = lax.rem(t, 2)
        nxt = 1 - par
        idx_copy(t + 1, nxt).wait()
        gather(nxt).start()
        gather_drain(par).wait()
        idx_copy(t + 2, par).start()

        base = lo8 + t * CHUNK
        end = base + CHUNK
        bdma = dma_base(t)

        def wcond(st):
            return jnp.logical_and(st[0] < NPW, ptr_at(st[0] + 1) <= end)

        def wbody(st):
            i = st[0]
            p0 = jnp.maximum(ptr_at(i), base)
            p1 = ptr_at(i + 1)
            acc = accum(st[1:], par, bdma, p0, p1)
            store_row(i, acc)
            return (i + 1,) + zero_acc

        st = lax.while_loop(wcond, wbody, carry)
        i = st[0]
        ic = jnp.minimum(i, NPW - 1)
        p0 = jnp.maximum(ptr_at(ic), base)
        p1 = jnp.where(i < NPW, jnp.minimum(ptr_at(ic + 1), end), p0)
        p0 = jnp.minimum(p0, p1)
        acc = accum(st[1:], par, bdma, p0, p1)
        return (i,) + acc

    st = lax.fori_loop(0, nt, chunk_body, (jnp.int32(0),) + zero_acc)

    gather_drain(0).wait()
    idx_copy(0, 1).wait()

    def fcond(st):
        return st[0] < NPW

    def fbody(st):
        store_row(st[0], st[1:])
        return (st[0] + 1,) + zero_acc

    lax.while_loop(fcond, fbody, st)
    qflush(0, 1).wait()


@jax.jit
def _sc_gnn(xw, ptr_pad, edge_pad):
    kern = pl.kernel(
        _sc_body,
        out_type=jax.ShapeDtypeStruct((N, D_OUT), jnp.float32),
        mesh=plsc.VectorSubcoreMesh(core_axis_name="c", subcore_axis_name="s"),
        compiler_params=pltpu.CompilerParams(needs_layout_passes=False),
        scratch_types=[
            pltpu.VMEM((PBUF,), jnp.int32),
            pltpu.VMEM((2 * CHUNK,), jnp.int32),
            pltpu.VMEM((2, CHUNK, D_OUT // 2), jnp.uint32),
            pltpu.VMEM((2, QH, D_OUT), jnp.float32),
            pltpu.SemaphoreType.DMA((2,)),
            pltpu.SemaphoreType.DMA((2,)),
            pltpu.SemaphoreType.DMA((2,)),
        ],
    )
    return kern(xw, ptr_pad, edge_pad)


def kernel(X, nodePointer, edgeList, blockPartition, edgeToColumn, edgeToRow,
           adj_coo, block_num, g_nodes, weights):
    XW = _matmul(X, weights)
    return _sc_gnn(XW, nodePointer, edgeList)
